# Initial kernel scaffold; baseline (speedup 1.0000x reference)
import jax
import jax.numpy as jnp
from jax.experimental import pallas as pl


def kernel(hidden_states, positions, kv_cache, seq_lens, slot_mapping, ln1_w, ln2_w, Wq, Wkv, Wo, Wg, w1, w2, Wse, Wsd):
    raise NotImplementedError("write your pallas kernel here")


# speedup vs baseline: 1.8647x; 1.8647x over previous
import jax
import jax.numpy as jnp
from jax.experimental import pallas as pl
from jax.experimental.pallas import tpu as pltpu

B = 16
HID = 2048
NH = 16
NOPE = 64
ROPE = 32
VD = 64
QD = NOPE + ROPE
E = 8
INTER = 1408
KV = 2048
EPS = 1e-06
SCALE = QD ** -0.5

HD = NH * NOPE
BLK = 256
NBLK = KV // BLK
CHUNK = 128
NC = INTER // CHUNK
E2 = E // 2

_F32 = jnp.float32
_DN = (((1,), (1,)), ((), ()))
_DN0 = (((1,), (0,)), ((), ()))


def _rms(x, w):
    return x * jax.lax.rsqrt(jnp.mean(x * x, axis=-1, keepdims=True) + EPS) * w


def _pre_body(x_ref, w_ref, wq_ref, wkv_ref, q_ref, kv_ref):
    xn = _rms(x_ref[...], w_ref[...])
    q_ref[...] = jax.lax.dot_general(xn, wq_ref[...], _DN,
                                     preferred_element_type=_F32)
    kv_ref[...] = jax.lax.dot_general(xn, wkv_ref[...], _DN,
                                      preferred_element_type=_F32)


def _attn_body(sl_ref, q_ref, kc_ref, vc_ref, kb_ref, vb_ref, s_ref, st_ref,
               o_ref, acc, m, l):
    b = pl.program_id(0)
    j = pl.program_id(1)
    ncache = sl_ref[b] - 1
    nb = (ncache + BLK - 1) // BLK

    q = q_ref[...].reshape(1, HD)

    @pl.when(j == 0)
    def _init():
        s_cur = jax.lax.dot_general(q * kc_ref[...].reshape(1, HD), s_ref[...],
                                    _DN0, preferred_element_type=_F32)
        m[...] = s_cur
        l[...] = jnp.ones_like(s_cur)
        acc[...] = vc_ref[...].reshape(1, HD)

    @pl.when(j < nb)
    def _block():
        kb = kb_ref[...].reshape(BLK, HD)
        vb = vb_ref[...].reshape(BLK, HD)
        scores = jax.lax.dot_general(kb * q, s_ref[...], _DN0,
                                     preferred_element_type=_F32)
        pos = j * BLK + jax.lax.broadcasted_iota(jnp.int32, (BLK, NH), 0)
        scores = jnp.where(pos < ncache, scores, -1e30)
        bm = jnp.max(scores, axis=0, keepdims=True)
        new_m = jnp.maximum(m[...], bm)
        corr = jnp.exp(m[...] - new_m)
        p = jnp.exp(scores - new_m)
        l[...] = l[...] * corr + jnp.sum(p, axis=0, keepdims=True)
        m[...] = new_m
        pe = jax.lax.dot_general(p, st_ref[...], _DN0,
                                 preferred_element_type=_F32)
        ce = jax.lax.dot_general(corr, st_ref[...], _DN0,
                                 preferred_element_type=_F32)
        acc[...] = acc[...] * ce + jnp.sum(pe * vb, axis=0, keepdims=True)

    @pl.when(j == NBLK - 1)
    def _fin():
        le = jax.lax.dot_general(l[...], st_ref[...], _DN0,
                                 preferred_element_type=_F32)
        o_ref[...] = (acc[...] / le).reshape(1, 1, HD)


def _mid_body(ao_ref, x_ref, wo_ref, wn_ref, wg_ref, wse_ref, wsd_ref,
              h2_ref, wt_ref, base_ref):
    attn_res = jax.lax.dot_general(ao_ref[...], wo_ref[...], _DN,
                                   preferred_element_type=_F32)
    resid = x_ref[...] + attn_res
    h2 = _rms(resid, wn_ref[...])
    h2_ref[...] = h2
    logits = jax.lax.dot_general(h2, wg_ref[...], _DN,
                                 preferred_element_type=_F32)
    ii = jax.lax.broadcasted_iota(jnp.int32, (B, E), 1)
    m1 = jnp.max(logits, axis=1, keepdims=True)
    i1 = jnp.min(jnp.where(logits == m1, ii, E), axis=1, keepdims=True)
    masked = jnp.where(ii == i1, -jnp.inf, logits)
    m2 = jnp.max(masked, axis=1, keepdims=True)
    i2 = jnp.min(jnp.where(masked == m2, ii, E), axis=1, keepdims=True)
    e2 = jnp.exp(m2 - m1)
    d = 1.0 + e2
    wmat = jnp.where(ii == i1, 1.0, 0.0) / d + jnp.where(ii == i2, e2, 0.0) / d
    wt_ref[...] = wmat.T
    su = jax.lax.dot_general(h2, wse_ref[...], _DN,
                             preferred_element_type=_F32)
    sg = su[:, :INTER]
    uu = su[:, INTER:]
    act = sg * jax.nn.sigmoid(sg) * uu
    shared = jax.lax.dot_general(act, wsd_ref[...], _DN,
                                 preferred_element_type=_F32)
    base_ref[...] = resid + shared


def _moe_body(x_ref, wt_ref, w1g_ref, w1u_ref, w2_ref, o_ref):
    e = pl.program_id(1)
    c = pl.program_id(2)

    @pl.when((e == 0) & (c == 0))
    def _zero():
        o_ref[...] = jnp.zeros_like(o_ref)

    x = x_ref[...]
    g = jax.lax.dot_general(x, w1g_ref[...].reshape(CHUNK, HID), _DN,
                            preferred_element_type=_F32)
    u = jax.lax.dot_general(x, w1u_ref[...].reshape(CHUNK, HID), _DN,
                            preferred_element_type=_F32)
    wcol = wt_ref[...].reshape(B, 1)
    act = g * jax.nn.sigmoid(g) * u * wcol
    o_ref[...] += jax.lax.dot_general(
        act, w2_ref[...].reshape(HID, CHUNK), _DN,
        preferred_element_type=_F32)[None]


def kernel(hidden_states, positions, kv_cache, seq_lens, slot_mapping,
           ln1_w, ln2_w, Wq, Wkv, Wo, Wg, w1, w2, Wse, Wsd):
    x = hidden_states[:, 0, :]
    sl = jnp.maximum(seq_lens, 1).astype(jnp.int32)
    kvc = kv_cache.reshape(2, B, KV, HD)

    q_full, kv_full = pl.pallas_call(
        _pre_body,
        out_shape=[jax.ShapeDtypeStruct((B, NH * QD), _F32),
                   jax.ShapeDtypeStruct((B, NH * (NOPE + VD) + ROPE), _F32)],
    )(x, ln1_w.reshape(1, HID), Wq, Wkv)

    q_nope = q_full.reshape(B, NH, QD)[:, :, :NOPE].reshape(B, 1, HD) * SCALE
    k_cur = kv_full[:, :HD].reshape(B, 1, HD)
    v_cur = kv_full[:, HD:2 * HD].reshape(B, 1, HD)

    rows = jax.lax.broadcasted_iota(jnp.int32, (HD, NH), 0) // NOPE
    cols = jax.lax.broadcasted_iota(jnp.int32, (HD, NH), 1)
    S = (rows == cols).astype(_F32)
    ST = S.T

    def _kv_idx(part):
        def idx(b, j, sl_ref):
            nb = (sl_ref[b] - 1 + BLK - 1) // BLK
            return (part, b, jnp.minimum(j, jnp.maximum(nb - 1, 0)), 0)
        return idx

    attn_out = pl.pallas_call(
        _attn_body,
        grid_spec=pltpu.PrefetchScalarGridSpec(
            num_scalar_prefetch=1,
            grid=(B, NBLK),
            in_specs=[
                pl.BlockSpec((1, 1, HD), lambda b, j, s: (b, 0, 0)),
                pl.BlockSpec((1, 1, HD), lambda b, j, s: (b, 0, 0)),
                pl.BlockSpec((1, 1, HD), lambda b, j, s: (b, 0, 0)),
                pl.BlockSpec((1, 1, BLK, HD), _kv_idx(0)),
                pl.BlockSpec((1, 1, BLK, HD), _kv_idx(1)),
                pl.BlockSpec((HD, NH), lambda b, j, s: (0, 0)),
                pl.BlockSpec((NH, HD), lambda b, j, s: (0, 0)),
            ],
            out_specs=pl.BlockSpec((1, 1, HD), lambda b, j, s: (b, 0, 0)),
            scratch_shapes=[pltpu.VMEM((1, HD), _F32),
                            pltpu.VMEM((1, NH), _F32),
                            pltpu.VMEM((1, NH), _F32)],
        ),
        out_shape=jax.ShapeDtypeStruct((B, 1, HD), _F32),
        compiler_params=pltpu.CompilerParams(
            dimension_semantics=("parallel", "arbitrary")),
    )(sl, q_nope, k_cur, v_cur, kvc, kvc, S, ST)

    h2, wt, base = pl.pallas_call(
        _mid_body,
        out_shape=[jax.ShapeDtypeStruct((B, HID), _F32),
                   jax.ShapeDtypeStruct((E, B), _F32),
                   jax.ShapeDtypeStruct((B, HID), _F32)],
    )(attn_out.reshape(B, HD), x, Wo, ln2_w.reshape(1, HID), Wg, Wse, Wsd)

    wt3 = wt[:, :, None]

    moe = pl.pallas_call(
        _moe_body,
        grid=(2, E2, NC),
        in_specs=[
            pl.BlockSpec((B, HID), lambda p, e, c: (0, 0)),
            pl.BlockSpec((1, B, 1), lambda p, e, c: (p * E2 + e, 0, 0)),
            pl.BlockSpec((1, CHUNK, HID), lambda p, e, c: (p * E2 + e, c, 0)),
            pl.BlockSpec((1, CHUNK, HID),
                         lambda p, e, c: (p * E2 + e, NC + c, 0)),
            pl.BlockSpec((1, HID, CHUNK), lambda p, e, c: (p * E2 + e, 0, c)),
        ],
        out_specs=pl.BlockSpec((1, B, HID), lambda p, e, c: (p, 0, 0)),
        out_shape=jax.ShapeDtypeStruct((2, B, HID), _F32),
        compiler_params=pltpu.CompilerParams(
            dimension_semantics=("parallel", "arbitrary", "arbitrary")),
    )(h2, wt3, w1, w1, w2)

    out = base + moe[0] + moe[1]
    return out[:, None, :]
